# Initial kernel scaffold; baseline (speedup 1.0000x reference)
import jax
import jax.numpy as jnp
from jax.experimental import pallas as pl


def kernel(edge_type, att_rc, att_rp, type_table, W_rc, b_rc, W_rp, b_rp):
    raise NotImplementedError("write your pallas kernel here")


# speedup vs baseline: 2.2909x; 2.2909x over previous
import functools

import jax
import jax.numpy as jnp
from jax import lax
from jax.experimental import pallas as pl

E = 320000
HID = 128
N_EDGE_TYPE = 16
BLOCK = 3200
NB = E // BLOCK


def _body(et_ref, rc_ref, rp_ref, tbl_ref, wrc_ref, brc_ref, wrp_ref,
          brp_ref, out_ref):
    et = et_ref[0, 0, :]
    onehot = (et[:, None] == lax.broadcasted_iota(jnp.int32, (1, N_EDGE_TYPE), 1)
              ).astype(jnp.float32)
    tbl2 = tbl_ref[...] + brc_ref[...] + brp_ref[...]
    acc = jnp.dot(onehot, tbl2, preferred_element_type=jnp.float32)
    rc = rc_ref[...]
    rp = rp_ref[...]
    acc = acc + rc[:, 0:1] * wrc_ref[0:1, :] + rc[:, 1:2] * wrc_ref[1:2, :]
    acc = acc + (rp[:, 0:1] * wrp_ref[0:1, :] + rp[:, 1:2] * wrp_ref[1:2, :]
                 + rp[:, 2:3] * wrp_ref[2:3, :])
    out_ref[...] = acc


@jax.jit
def kernel(edge_type, att_rc, att_rp, type_table, W_rc, b_rc, W_rp, b_rp):
    et3 = edge_type.astype(jnp.int32).reshape(NB, 1, BLOCK)
    brc = b_rc.reshape(1, HID)
    brp = b_rp.reshape(1, HID)
    grid = (NB,)
    return pl.pallas_call(
        _body,
        grid=grid,
        in_specs=[
            pl.BlockSpec((1, 1, BLOCK), lambda i: (i, 0, 0)),
            pl.BlockSpec((BLOCK, 2), lambda i: (i, 0)),
            pl.BlockSpec((BLOCK, 3), lambda i: (i, 0)),
            pl.BlockSpec((N_EDGE_TYPE, HID), lambda i: (0, 0)),
            pl.BlockSpec((2, HID), lambda i: (0, 0)),
            pl.BlockSpec((1, HID), lambda i: (0, 0)),
            pl.BlockSpec((3, HID), lambda i: (0, 0)),
            pl.BlockSpec((1, HID), lambda i: (0, 0)),
        ],
        out_specs=pl.BlockSpec((BLOCK, HID), lambda i: (i, 0)),
        out_shape=jax.ShapeDtypeStruct((E, HID), jnp.float32),
    )(et3, att_rc, att_rp, type_table, W_rc, brc, W_rp, brp)


# speedup vs baseline: 2.9833x; 1.3022x over previous
import functools

import jax
import jax.numpy as jnp
from jax import lax
from jax.experimental import pallas as pl

E = 320000
HID = 128
N_EDGE_TYPE = 16
BLOCK = 6400
NB = E // BLOCK


def _body(et_ref, rc_ref, rp_ref, tbl_ref, wrc_ref, brc_ref, wrp_ref,
          brp_ref, out_ref):
    et = et_ref[0, 0, :]
    onehot = (et[:, None] == lax.broadcasted_iota(jnp.int32, (1, N_EDGE_TYPE), 1)
              ).astype(jnp.float32)
    tbl2 = tbl_ref[...] + brc_ref[...] + brp_ref[...]
    x = jnp.concatenate([onehot, rc_ref[...], rp_ref[...]], axis=1)
    w = jnp.concatenate([tbl2, wrc_ref[...], wrp_ref[...]], axis=0)
    out_ref[...] = jax.lax.dot(x, w, preferred_element_type=jnp.float32)


@jax.jit
def kernel(edge_type, att_rc, att_rp, type_table, W_rc, b_rc, W_rp, b_rp):
    et3 = edge_type.astype(jnp.int32).reshape(NB, 1, BLOCK)
    brc = b_rc.reshape(1, HID)
    brp = b_rp.reshape(1, HID)
    grid = (NB,)
    return pl.pallas_call(
        _body,
        grid=grid,
        in_specs=[
            pl.BlockSpec((1, 1, BLOCK), lambda i: (i, 0, 0)),
            pl.BlockSpec((BLOCK, 2), lambda i: (i, 0)),
            pl.BlockSpec((BLOCK, 3), lambda i: (i, 0)),
            pl.BlockSpec((N_EDGE_TYPE, HID), lambda i: (0, 0)),
            pl.BlockSpec((2, HID), lambda i: (0, 0)),
            pl.BlockSpec((1, HID), lambda i: (0, 0)),
            pl.BlockSpec((3, HID), lambda i: (0, 0)),
            pl.BlockSpec((1, HID), lambda i: (0, 0)),
        ],
        out_specs=pl.BlockSpec((BLOCK, HID), lambda i: (i, 0)),
        out_shape=jax.ShapeDtypeStruct((E, HID), jnp.float32),
    )(et3, att_rc, att_rp, type_table, W_rc, brc, W_rp, brp)


# speedup vs baseline: 3.1761x; 1.0646x over previous
import functools

import jax
import jax.numpy as jnp
from jax import lax
from jax.experimental import pallas as pl

E = 320000
HID = 128
N_EDGE_TYPE = 16
BLOCK = 6400
NB = E // BLOCK


def _body(et_ref, rc_ref, rp_ref, tbl_ref, wrc_ref, brc_ref, wrp_ref,
          brp_ref, out_ref):
    et = et_ref[0, 0, :]
    onehot = (et[:, None] == lax.broadcasted_iota(jnp.int32, (1, N_EDGE_TYPE), 1)
              ).astype(jnp.float32)
    tbl2 = tbl_ref[...] + brc_ref[...] + brp_ref[...]
    acc = jnp.dot(onehot, tbl2, preferred_element_type=jnp.float32)
    acc = acc + jnp.dot(rc_ref[...], wrc_ref[...],
                        preferred_element_type=jnp.float32)
    acc = acc + jnp.dot(rp_ref[...], wrp_ref[...],
                        preferred_element_type=jnp.float32)
    out_ref[...] = acc


@jax.jit
def kernel(edge_type, att_rc, att_rp, type_table, W_rc, b_rc, W_rp, b_rp):
    et3 = edge_type.astype(jnp.int32).reshape(NB, 1, BLOCK)
    brc = b_rc.reshape(1, HID)
    brp = b_rp.reshape(1, HID)
    grid = (NB,)
    return pl.pallas_call(
        _body,
        grid=grid,
        in_specs=[
            pl.BlockSpec((1, 1, BLOCK), lambda i: (i, 0, 0)),
            pl.BlockSpec((BLOCK, 2), lambda i: (i, 0)),
            pl.BlockSpec((BLOCK, 3), lambda i: (i, 0)),
            pl.BlockSpec((N_EDGE_TYPE, HID), lambda i: (0, 0)),
            pl.BlockSpec((2, HID), lambda i: (0, 0)),
            pl.BlockSpec((1, HID), lambda i: (0, 0)),
            pl.BlockSpec((3, HID), lambda i: (0, 0)),
            pl.BlockSpec((1, HID), lambda i: (0, 0)),
        ],
        out_specs=pl.BlockSpec((BLOCK, HID), lambda i: (i, 0)),
        out_shape=jax.ShapeDtypeStruct((E, HID), jnp.float32),
    )(et3, att_rc, att_rp, type_table, W_rc, brc, W_rp, brp)


# speedup vs baseline: 3.2397x; 1.0200x over previous
import functools

import jax
import jax.numpy as jnp
from jax import lax
from jax.experimental import pallas as pl

E = 320000
HID = 128
N_EDGE_TYPE = 16
BLOCK = 16000
NB = E // BLOCK


def _body(et_ref, rc_ref, rp_ref, tbl_ref, wrc_ref, brc_ref, wrp_ref,
          brp_ref, out_ref):
    et = et_ref[0, 0, :]
    onehot = (et[:, None] == lax.broadcasted_iota(jnp.int32, (1, N_EDGE_TYPE), 1)
              ).astype(jnp.float32)
    tbl2 = tbl_ref[...] + brc_ref[...] + brp_ref[...]
    acc = jnp.dot(onehot, tbl2, preferred_element_type=jnp.float32)
    acc = acc + jnp.dot(rc_ref[...], wrc_ref[...],
                        preferred_element_type=jnp.float32)
    acc = acc + jnp.dot(rp_ref[...], wrp_ref[...],
                        preferred_element_type=jnp.float32)
    out_ref[...] = acc


@jax.jit
def kernel(edge_type, att_rc, att_rp, type_table, W_rc, b_rc, W_rp, b_rp):
    et3 = edge_type.astype(jnp.int32).reshape(NB, 1, BLOCK)
    brc = b_rc.reshape(1, HID)
    brp = b_rp.reshape(1, HID)
    grid = (NB,)
    return pl.pallas_call(
        _body,
        grid=grid,
        in_specs=[
            pl.BlockSpec((1, 1, BLOCK), lambda i: (i, 0, 0)),
            pl.BlockSpec((BLOCK, 2), lambda i: (i, 0)),
            pl.BlockSpec((BLOCK, 3), lambda i: (i, 0)),
            pl.BlockSpec((N_EDGE_TYPE, HID), lambda i: (0, 0)),
            pl.BlockSpec((2, HID), lambda i: (0, 0)),
            pl.BlockSpec((1, HID), lambda i: (0, 0)),
            pl.BlockSpec((3, HID), lambda i: (0, 0)),
            pl.BlockSpec((1, HID), lambda i: (0, 0)),
        ],
        out_specs=pl.BlockSpec((BLOCK, HID), lambda i: (i, 0)),
        out_shape=jax.ShapeDtypeStruct((E, HID), jnp.float32),
    )(et3, att_rc, att_rp, type_table, W_rc, brc, W_rp, brp)
